# Initial kernel scaffold; baseline (speedup 1.0000x reference)
import jax
import jax.numpy as jnp
from jax.experimental import pallas as pl


def kernel(h, edge_m, norm, edge_index, W, b, ln_g, ln_b):
    raise NotImplementedError("write your pallas kernel here")


# speedup vs baseline: 3.3369x; 3.3369x over previous
import functools

import jax
import jax.numpy as jnp
from jax import lax
from jax.experimental import pallas as pl
from jax.experimental.pallas import tpu as pltpu
from jax.experimental.pallas import tpu_sc as plsc

N_NODES = 100000
N_EDGES = 3200000
IN_FEATS = 128
ADDED = 24
OUT_FEATS = 128

FEATS0 = 16
FEATS1 = 8
LANES = 128
IDX_ROWS = N_EDGES // LANES
CHUNK_ROWS = 8
CHUNK_E = CHUNK_ROWS * LANES
N_FULL_CHUNKS = IDX_ROWS // CHUNK_ROWS
TAIL_ROWS = IDX_ROWS - N_FULL_CHUNKS * CHUNK_ROWS
N_TILES = 16
ROWS_PER_TILE = N_NODES // N_TILES
MAX_K = (N_FULL_CHUNKS + N_TILES - 1) // N_TILES


def _sc_segment_sum(edge_m, dst2d, zeros):
    mesh = plsc.VectorSubcoreMesh(core_axis_name="c", subcore_axis_name="s")

    @functools.partial(
        pl.kernel,
        out_type=jax.ShapeDtypeStruct((N_NODES, ADDED), jnp.float32),
        mesh=mesh,
        scratch_types=[
            pltpu.VMEM((CHUNK_ROWS, LANES), jnp.int32),
            pltpu.VMEM((CHUNK_E, FEATS0), jnp.float32),
            pltpu.VMEM_SHARED((N_NODES, FEATS0), jnp.float32),
        ],
        compiler_params=pltpu.CompilerParams(use_tc_tiling_on_sc=False),
    )
    def run(em_hbm, dst_hbm, zero_hbm, out_hbm, idx_v, rows_v, acc):
        c = lax.axis_index("c")
        t = lax.axis_index("s")
        node0 = t * ROWS_PER_TILE

        pltpu.sync_copy(zero_hbm,
                        acc.at[pl.ds(node0, ROWS_PER_TILE), :])
        @pl.when(c == 1)
        def _():
            pltpu.sync_copy(zero_hbm.at[pl.ds(0, CHUNK_E), pl.ds(0, FEATS1)],
                            rows_v.at[:, pl.ds(FEATS1, FEATS1)])

        plsc.subcore_barrier()

        def do_chunk(row0, nrows):
            ne = nrows * LANES
            e0 = row0 * LANES
            pltpu.sync_copy(dst_hbm.at[pl.ds(row0, nrows), :],
                            idx_v.at[pl.ds(0, nrows), :])

            @pl.when(c == 0)
            def _():
                pltpu.sync_copy(em_hbm.at[pl.ds(e0, ne), pl.ds(0, FEATS0)],
                                rows_v.at[pl.ds(0, ne), :])

            @pl.when(c == 1)
            def _():
                pltpu.sync_copy(em_hbm.at[pl.ds(e0, ne), pl.ds(FEATS0, FEATS1)],
                                rows_v.at[pl.ds(0, ne), pl.ds(0, FEATS1)])

            for j in range(nrows):
                pltpu.sync_copy(rows_v.at[pl.ds(j * LANES, LANES), :],
                                acc.at[idx_v.at[j]], add=True)

        def body(k, carry):
            chunk = t + k * N_TILES

            @pl.when(chunk < N_FULL_CHUNKS)
            def _():
                do_chunk(chunk * CHUNK_ROWS, CHUNK_ROWS)

            return carry

        lax.fori_loop(0, MAX_K, body, 0)

        if TAIL_ROWS:
            @pl.when(t == N_TILES - 1)
            def _():
                do_chunk(N_FULL_CHUNKS * CHUNK_ROWS, TAIL_ROWS)

        plsc.subcore_barrier()

        @pl.when(c == 0)
        def _():
            pltpu.sync_copy(acc.at[pl.ds(node0, ROWS_PER_TILE), :],
                            out_hbm.at[pl.ds(node0, ROWS_PER_TILE),
                                       pl.ds(0, FEATS0)])

        @pl.when(c == 1)
        def _():
            pltpu.sync_copy(acc.at[pl.ds(node0, ROWS_PER_TILE),
                                   pl.ds(0, FEATS1)],
                            out_hbm.at[pl.ds(node0, ROWS_PER_TILE),
                                       pl.ds(FEATS0, FEATS1)])

    return run(edge_m, dst2d, zeros)


def _tc_dense(h, ah, norm, w1t, w2t, b2, g2, be2):
    BR = 1000
    grid = N_NODES // BR

    def body(h_ref, ah_ref, n_ref, w1_ref, w2_ref, b_ref, g_ref, be_ref,
             o_ref):
        x = (jnp.dot(h_ref[...], w1_ref[...],
                     preferred_element_type=jnp.float32)
             + jnp.dot(ah_ref[...] * n_ref[...], w2_ref[...],
                       preferred_element_type=jnp.float32)
             + b_ref[...])
        mu = jnp.mean(x, axis=1, keepdims=True)
        xc = x - mu
        var = jnp.mean(xc * xc, axis=1, keepdims=True)
        y = xc * lax.rsqrt(var + 1e-5) * g_ref[...] + be_ref[...]
        o_ref[...] = jnp.maximum(y, 0.0)

    return pl.pallas_call(
        body,
        grid=(grid,),
        in_specs=[
            pl.BlockSpec((BR, IN_FEATS), lambda i: (i, 0)),
            pl.BlockSpec((BR, ADDED), lambda i: (i, 0)),
            pl.BlockSpec((BR, 1), lambda i: (i, 0)),
            pl.BlockSpec((IN_FEATS, OUT_FEATS), lambda i: (0, 0)),
            pl.BlockSpec((ADDED, OUT_FEATS), lambda i: (0, 0)),
            pl.BlockSpec((1, OUT_FEATS), lambda i: (0, 0)),
            pl.BlockSpec((1, OUT_FEATS), lambda i: (0, 0)),
            pl.BlockSpec((1, OUT_FEATS), lambda i: (0, 0)),
        ],
        out_specs=pl.BlockSpec((BR, OUT_FEATS), lambda i: (i, 0)),
        out_shape=jax.ShapeDtypeStruct((N_NODES, OUT_FEATS), jnp.float32),
    )(h, ah, norm, w1t, w2t, b2, g2, be2)


def kernel(h, edge_m, norm, edge_index, W, b, ln_g, ln_b):
    dst2d = edge_index[1].astype(jnp.int32).reshape(IDX_ROWS, LANES)
    zeros = jnp.zeros((ROWS_PER_TILE, FEATS0), jnp.float32)
    ah = _sc_segment_sum(edge_m, dst2d, zeros)
    w1t = W[:, :IN_FEATS].T
    w2t = W[:, IN_FEATS:].T
    return _tc_dense(h, ah, norm, w1t, w2t, b.reshape(1, -1),
                     ln_g.reshape(1, -1), ln_b.reshape(1, -1))


# speedup vs baseline: 3.8123x; 1.1425x over previous
import functools

import jax
import jax.numpy as jnp
from jax import lax
from jax.experimental import pallas as pl
from jax.experimental.pallas import tpu as pltpu
from jax.experimental.pallas import tpu_sc as plsc

N_NODES = 100000
N_EDGES = 3200000
IN_FEATS = 128
ADDED = 24
OUT_FEATS = 128

FEATS0 = 16
FEATS1 = 8
LANES = 128
IDX_ROWS = N_EDGES // LANES
CHUNK_ROWS = 4
CHUNK_E = CHUNK_ROWS * LANES
N_CHUNKS = IDX_ROWS // CHUNK_ROWS
N_TILES = 16
ROWS_PER_TILE = N_NODES // N_TILES
NBUF = 3
N_SLOTS = 393
N_SUPER = N_SLOTS // NBUF


def _sc_segment_sum(edge_m, dst2d, zeros):
    mesh = plsc.VectorSubcoreMesh(core_axis_name="c", subcore_axis_name="s")

    @functools.partial(
        pl.kernel,
        out_type=jax.ShapeDtypeStruct((N_NODES, ADDED), jnp.float32),
        mesh=mesh,
        scratch_types=[
            pltpu.VMEM((NBUF, CHUNK_ROWS, LANES), jnp.int32),
            pltpu.VMEM((NBUF, CHUNK_E, FEATS0), jnp.float32),
            pltpu.VMEM_SHARED((N_NODES, FEATS0), jnp.float32),
            pltpu.SemaphoreType.DMA,
            pltpu.SemaphoreType.DMA,
            pltpu.SemaphoreType.DMA,
            pltpu.SemaphoreType.DMA,
            pltpu.SemaphoreType.DMA,
            pltpu.SemaphoreType.DMA,
        ],
        compiler_params=pltpu.CompilerParams(use_tc_tiling_on_sc=False),
    )
    def run(em_hbm, dst_hbm, zero_hbm, out_hbm, idx_v, rows_v, acc,
            l0, l1, l2, s0, s1, s2):
        c = lax.axis_index("c")
        t = lax.axis_index("s")
        node0 = t * ROWS_PER_TILE
        lsem = (l0, l1, l2)
        ssem = (s0, s1, s2)

        pltpu.sync_copy(zero_hbm,
                        acc.at[pl.ds(node0, ROWS_PER_TILE), :])
        @pl.when(c == 1)
        def _():
            for b in range(NBUF):
                pltpu.sync_copy(
                    zero_hbm.at[pl.ds(0, CHUNK_E), pl.ds(0, FEATS1)],
                    rows_v.at[b].at[:, pl.ds(FEATS1, FEATS1)])

        plsc.subcore_barrier()

        def chunk_of(n):
            return t + n * N_TILES

        def load_descs(b, n):
            ch = chunk_of(n)
            row0 = ch * CHUNK_ROWS
            e0 = ch * CHUNK_E
            idx_d = (dst_hbm.at[pl.ds(row0, CHUNK_ROWS), :], idx_v.at[b],
                     lsem[b])
            row_d0 = (em_hbm.at[pl.ds(e0, CHUNK_E), pl.ds(0, FEATS0)],
                      rows_v.at[b], lsem[b])
            row_d1 = (em_hbm.at[pl.ds(e0, CHUNK_E), pl.ds(FEATS0, FEATS1)],
                      rows_v.at[b].at[:, pl.ds(0, FEATS1)], lsem[b])
            return idx_d, row_d0, row_d1

        def start_loads(b, n):
            idx_d, row_d0, row_d1 = load_descs(b, n)
            pltpu.async_copy(*idx_d)

            @pl.when(c == 0)
            def _():
                pltpu.async_copy(*row_d0)

            @pl.when(c == 1)
            def _():
                pltpu.async_copy(*row_d1)

        def wait_loads(b, n):
            idx_d, row_d0, row_d1 = load_descs(b, n)
            pltpu.make_async_copy(*idx_d).wait()

            @pl.when(c == 0)
            def _():
                pltpu.make_async_copy(*row_d0).wait()

            @pl.when(c == 1)
            def _():
                pltpu.make_async_copy(*row_d1).wait()

        def sct_descs(b, j):
            return (rows_v.at[b].at[pl.ds(j * LANES, LANES), :],
                    acc.at[idx_v.at[b].at[j]], ssem[b])

        def start_scts(b):
            for j in range(CHUNK_ROWS):
                pltpu.async_copy(*sct_descs(b, j), add=True)

        def wait_scts(b):
            for j in range(CHUNK_ROWS):
                pltpu.make_async_copy(*sct_descs(b, j)).wait()

        for n0 in range(NBUF - 1):
            @pl.when(chunk_of(n0) < N_CHUNKS)
            def _(n0=n0):
                start_loads(n0, n0)

        def superstep(s, carry):
            for b in range(NBUF):
                n = s * NBUF + b
                valid_n = chunk_of(n) < N_CHUNKS

                @pl.when(valid_n)
                def _(b=b, n=n):
                    wait_loads(b, n)
                    start_scts(b)

                b2 = (b + 2) % NBUF

                @pl.when((n >= 1) & (chunk_of(n - 1) < N_CHUNKS))
                def _(b2=b2):
                    wait_scts(b2)

                @pl.when(chunk_of(n + 2) < N_CHUNKS)
                def _(b2=b2, n=n):
                    start_loads(b2, n + 2)

            return carry

        lax.fori_loop(0, N_SUPER, superstep, 0)

        plsc.subcore_barrier()

        @pl.when(c == 0)
        def _():
            pltpu.sync_copy(acc.at[pl.ds(node0, ROWS_PER_TILE), :],
                            out_hbm.at[pl.ds(node0, ROWS_PER_TILE),
                                       pl.ds(0, FEATS0)])

        @pl.when(c == 1)
        def _():
            pltpu.sync_copy(acc.at[pl.ds(node0, ROWS_PER_TILE),
                                   pl.ds(0, FEATS1)],
                            out_hbm.at[pl.ds(node0, ROWS_PER_TILE),
                                       pl.ds(FEATS0, FEATS1)])

    return run(edge_m, dst2d, zeros)


def _tc_dense(h, ah, norm, w1t, w2t, b2, g2, be2):
    BR = 1000
    grid = N_NODES // BR

    def body(h_ref, ah_ref, n_ref, w1_ref, w2_ref, b_ref, g_ref, be_ref,
             o_ref):
        x = (jnp.dot(h_ref[...], w1_ref[...],
                     preferred_element_type=jnp.float32)
             + jnp.dot(ah_ref[...] * n_ref[...], w2_ref[...],
                       preferred_element_type=jnp.float32)
             + b_ref[...])
        mu = jnp.mean(x, axis=1, keepdims=True)
        xc = x - mu
        var = jnp.mean(xc * xc, axis=1, keepdims=True)
        y = xc * lax.rsqrt(var + 1e-5) * g_ref[...] + be_ref[...]
        o_ref[...] = jnp.maximum(y, 0.0)

    return pl.pallas_call(
        body,
        grid=(grid,),
        in_specs=[
            pl.BlockSpec((BR, IN_FEATS), lambda i: (i, 0)),
            pl.BlockSpec((BR, ADDED), lambda i: (i, 0)),
            pl.BlockSpec((BR, 1), lambda i: (i, 0)),
            pl.BlockSpec((IN_FEATS, OUT_FEATS), lambda i: (0, 0)),
            pl.BlockSpec((ADDED, OUT_FEATS), lambda i: (0, 0)),
            pl.BlockSpec((1, OUT_FEATS), lambda i: (0, 0)),
            pl.BlockSpec((1, OUT_FEATS), lambda i: (0, 0)),
            pl.BlockSpec((1, OUT_FEATS), lambda i: (0, 0)),
        ],
        out_specs=pl.BlockSpec((BR, OUT_FEATS), lambda i: (i, 0)),
        out_shape=jax.ShapeDtypeStruct((N_NODES, OUT_FEATS), jnp.float32),
    )(h, ah, norm, w1t, w2t, b2, g2, be2)


def kernel(h, edge_m, norm, edge_index, W, b, ln_g, ln_b):
    dst2d = edge_index[1].astype(jnp.int32).reshape(IDX_ROWS, LANES)
    zeros = jnp.zeros((ROWS_PER_TILE, FEATS0), jnp.float32)
    ah = _sc_segment_sum(edge_m, dst2d, zeros)
    w1t = W[:, :IN_FEATS].T
    w2t = W[:, IN_FEATS:].T
    return _tc_dense(h, ah, norm, w1t, w2t, b.reshape(1, -1),
                     ln_g.reshape(1, -1), ln_b.reshape(1, -1))


# speedup vs baseline: 10.5862x; 2.7769x over previous
import functools

import jax
import jax.numpy as jnp
from jax import lax
from jax.experimental import pallas as pl
from jax.experimental.pallas import tpu as pltpu
from jax.experimental.pallas import tpu_sc as plsc

N_NODES = 100000
N_EDGES = 3200000
IN_FEATS = 128
ADDED = 24
OUT_FEATS = 128

FG = 8
N_GROUPS = ADDED // FG
LANES = 128
IDX_ROWS = N_EDGES // LANES
CHUNK_ROWS = 2
CHUNK_E = CHUNK_ROWS * LANES
N_CHUNKS = IDX_ROWS // CHUNK_ROWS
HALF_CHUNKS = N_CHUNKS // 2
N_TILES = 16
ROWS_PER_TILE = N_NODES // N_TILES
NBUF = 3
NPIECE = 8
N_SLOTS = 786
N_SUPER = N_SLOTS // 6


def _sc_segment_sum(em4d, dst2d, zeros):
    mesh = plsc.VectorSubcoreMesh(core_axis_name="c", subcore_axis_name="s")

    @functools.partial(
        pl.kernel,
        out_type=tuple(jax.ShapeDtypeStruct((N_NODES, FG), jnp.float32)
                       for _ in range(4)),
        mesh=mesh,
        scratch_types=[
            pltpu.VMEM((NBUF, CHUNK_ROWS, LANES), jnp.int32),
            pltpu.VMEM((NBUF, 2, CHUNK_ROWS, FG, LANES), jnp.float32),
            pltpu.VMEM((NPIECE, LANES, FG), jnp.float32),
            pltpu.VMEM_SHARED((N_NODES, FG), jnp.float32),
            pltpu.VMEM_SHARED((N_NODES, FG), jnp.float32),
        ] + [pltpu.SemaphoreType.DMA] * (NBUF + NPIECE),
        compiler_params=pltpu.CompilerParams(use_tc_tiling_on_sc=False,
                                             needs_layout_passes=False),
    )
    def run(em_hbm, dst_hbm, zero_hbm, out0_hbm, out1a_hbm, out1b_hbm,
            out2_hbm, idx_v, slab_v, rows_v, accp, accs, *sems):
        c = lax.axis_index("c")
        t = lax.axis_index("s")
        node0 = t * ROWS_PER_TILE
        lsem = sems[:NBUF]
        ssem = sems[NBUF:]

        pltpu.sync_copy(zero_hbm, accp.at[pl.ds(node0, ROWS_PER_TILE), :])
        pltpu.sync_copy(zero_hbm, accs.at[pl.ds(node0, ROWS_PER_TILE), :])
        plsc.subcore_barrier()

        ii = lax.broadcasted_iota(jnp.int32, (16,), 0)
        colv = [jnp.full((16,), f, jnp.int32) for f in range(FG)]

        def chunk_of(n):
            return t + n * N_TILES

        def valid(n):
            return chunk_of(n) < N_CHUNKS

        def sec(n):
            ch = chunk_of(n)
            return jnp.where(c == 0, ch < HALF_CHUNKS,
                             (ch >= HALF_CHUNKS) & (ch < N_CHUNKS))

        def load_descs(b, n):
            ch = chunk_of(n)
            row0 = ch * CHUNK_ROWS
            idx_d = (dst_hbm.at[pl.ds(row0, CHUNK_ROWS), :], idx_v.at[b],
                     lsem[b])
            p0_d = (em_hbm.at[0, pl.ds(row0, CHUNK_ROWS)],
                    slab_v.at[b, 0], lsem[b])
            p2_d = (em_hbm.at[2, pl.ds(row0, CHUNK_ROWS)],
                    slab_v.at[b, 0], lsem[b])
            s_d = (em_hbm.at[1, pl.ds(row0, CHUNK_ROWS)],
                   slab_v.at[b, 1], lsem[b])
            return idx_d, p0_d, p2_d, s_d

        def start_loads(b, n):
            idx_d, p0_d, p2_d, s_d = load_descs(b, n)
            pltpu.async_copy(*idx_d)

            @pl.when(c == 0)
            def _():
                pltpu.async_copy(*p0_d)

            @pl.when(c == 1)
            def _():
                pltpu.async_copy(*p2_d)

            @pl.when(sec(n))
            def _():
                pltpu.async_copy(*s_d)

        def wait_loads(b, n):
            idx_d, p0_d, _, s_d = load_descs(b, n)
            pltpu.make_async_copy(*idx_d).wait()
            pltpu.make_async_copy(*p0_d).wait()

            @pl.when(sec(n))
            def _():
                pltpu.make_async_copy(*s_d).wait()

        def sct_desc(p, b, r, grp):
            acc = accp if grp == 0 else accs
            return (rows_v.at[p], acc.at[idx_v.at[b, r]], ssem[p])

        def transpose_piece(b, grp, r, p):
            slab_r = slab_v.at[b, grp, r]

            def tr_body(h, carry):
                rv = ii + h * 16
                base = h * 16
                for f in range(FG):
                    v = slab_r[f, pl.ds(base, 16)]
                    plsc.store_scatter(rows_v.at[p], [rv, colv[f]], v)
                return carry

            lax.fori_loop(0, LANES // 16, tr_body, 0)

        def slot(n, b, q):
            qb = 4 * (1 - q)
            for r in range(CHUNK_ROWS):
                @pl.when((n >= 1) & valid(n - 1))
                def _(r=r):
                    pltpu.make_async_copy(*sct_desc(qb + r, 0, 0, 0)).wait()

                @pl.when((n >= 1) & valid(n - 1) & sec(n - 1))
                def _(r=r):
                    pltpu.make_async_copy(
                        *sct_desc(qb + 2 + r, 0, 0, 0)).wait()

            @pl.when(valid(n))
            def _():
                wait_loads(b, n)
                for r in range(CHUNK_ROWS):
                    p = 4 * q + r
                    transpose_piece(b, 0, r, p)
                    pltpu.async_copy(*sct_desc(p, b, r, 0), add=True)

            @pl.when(valid(n) & sec(n))
            def _():
                for r in range(CHUNK_ROWS):
                    p = 4 * q + 2 + r
                    transpose_piece(b, 1, r, p)
                    pltpu.async_copy(*sct_desc(p, b, r, 1), add=True)

            @pl.when(valid(n + 2))
            def _():
                start_loads((b + 2) % NBUF, n + 2)

        for n0 in range(2):
            @pl.when(valid(n0))
            def _(n0=n0):
                start_loads(n0, n0)

        def superstep(s, carry):
            for k in range(6):
                slot(s * 6 + k, k % 3, k % 2)
            return carry

        lax.fori_loop(0, N_SUPER, superstep, 0)

        plsc.subcore_barrier()

        nslice = pl.ds(node0, ROWS_PER_TILE)

        @pl.when(c == 0)
        def _():
            pltpu.sync_copy(accp.at[nslice, :], out0_hbm.at[nslice, :])
            pltpu.sync_copy(accs.at[nslice, :], out1a_hbm.at[nslice, :])

        @pl.when(c == 1)
        def _():
            pltpu.sync_copy(accp.at[nslice, :], out2_hbm.at[nslice, :])
            pltpu.sync_copy(accs.at[nslice, :], out1b_hbm.at[nslice, :])

    return run(em4d, dst2d, zeros)


def _tc_dense(h, g0, g1a, g1b, g2, norm, w1t, w20t, w21t, w22t, b2, gg2,
              be2):
    BR = 1000
    grid = N_NODES // BR

    def body(h_ref, g0_ref, g1a_ref, g1b_ref, g2_ref, n_ref, w1_ref,
             w20_ref, w21_ref, w22_ref, b_ref, g_ref, be_ref, o_ref):
        nb = n_ref[...]
        x = (jnp.dot(h_ref[...], w1_ref[...],
                     preferred_element_type=jnp.float32)
             + jnp.dot(g0_ref[...] * nb, w20_ref[...],
                       preferred_element_type=jnp.float32)
             + jnp.dot((g1a_ref[...] + g1b_ref[...]) * nb, w21_ref[...],
                       preferred_element_type=jnp.float32)
             + jnp.dot(g2_ref[...] * nb, w22_ref[...],
                       preferred_element_type=jnp.float32)
             + b_ref[...])
        mu = jnp.mean(x, axis=1, keepdims=True)
        xc = x - mu
        var = jnp.mean(xc * xc, axis=1, keepdims=True)
        y = xc * lax.rsqrt(var + 1e-5) * g_ref[...] + be_ref[...]
        o_ref[...] = jnp.maximum(y, 0.0)

    gspec = pl.BlockSpec((BR, FG), lambda i: (i, 0))
    wspec = pl.BlockSpec((FG, OUT_FEATS), lambda i: (0, 0))
    vspec = pl.BlockSpec((1, OUT_FEATS), lambda i: (0, 0))
    return pl.pallas_call(
        body,
        grid=(grid,),
        in_specs=[
            pl.BlockSpec((BR, IN_FEATS), lambda i: (i, 0)),
            gspec, gspec, gspec, gspec,
            pl.BlockSpec((BR, 1), lambda i: (i, 0)),
            pl.BlockSpec((IN_FEATS, OUT_FEATS), lambda i: (0, 0)),
            wspec, wspec, wspec,
            vspec, vspec, vspec,
        ],
        out_specs=pl.BlockSpec((BR, OUT_FEATS), lambda i: (i, 0)),
        out_shape=jax.ShapeDtypeStruct((N_NODES, OUT_FEATS), jnp.float32),
    )(h, g0, g1a, g1b, g2, norm, w1t, w20t, w21t, w22t, b2, gg2, be2)


def kernel(h, edge_m, norm, edge_index, W, b, ln_g, ln_b):
    dst2d = edge_index[1].astype(jnp.int32).reshape(IDX_ROWS, LANES)
    em4d = edge_m.T.reshape(N_GROUPS, FG, IDX_ROWS, LANES).transpose(
        0, 2, 1, 3)
    zeros = jnp.zeros((ROWS_PER_TILE, FG), jnp.float32)
    g0, g1a, g1b, g2 = _sc_segment_sum(em4d, dst2d, zeros)
    w1t = W[:, :IN_FEATS].T
    w20t = W[:, IN_FEATS:IN_FEATS + FG].T
    w21t = W[:, IN_FEATS + FG:IN_FEATS + 2 * FG].T
    w22t = W[:, IN_FEATS + 2 * FG:].T
    return _tc_dense(h, g0, g1a, g1b, g2, norm, w1t, w20t, w21t, w22t,
                     b.reshape(1, -1), ln_g.reshape(1, -1),
                     ln_b.reshape(1, -1))
